# Initial kernel scaffold; baseline (speedup 1.0000x reference)
import jax
import jax.numpy as jnp
from jax.experimental import pallas as pl


def kernel(seq1, seq2, edge_index1, edge_weight1, edge_index2, edge_weight2, params, sparse):
    raise NotImplementedError("write your pallas kernel here")


# speedup vs baseline: 3.9610x; 3.9610x over previous
import functools

import jax
import jax.numpy as jnp
from jax import lax
from jax.experimental import pallas as pl
from jax.experimental.pallas import tpu as pltpu
from jax.experimental.pallas import tpu_sc as plsc

_F = 128
_TILES = 16
_B = 128
_LANES = 16


def _seg_body(n_nodes, ep_per_set, ept, seq1_hbm, seq2_hbm, src_hbm, dst_hbm,
              w_hbm, zeros_hbm, out_hbm, acc, rows_v, src_v, dst_v, w_v, sem):
    c = lax.axis_index("c")
    s = lax.axis_index("s")
    rows_per_tile = n_nodes // _TILES
    nb = ept // _B
    my_rows = pl.ds(s * rows_per_tile, rows_per_tile)
    for sq, seq_hbm in enumerate((seq1_hbm, seq2_hbm)):
        pltpu.sync_copy(zeros_hbm.at[pl.ds(0, rows_per_tile)], acc.at[my_rows])
        plsc.subcore_barrier()

        def batch_body(b, carry):
            base = pl.multiple_of(c * ep_per_set + s * ept + b * _B, _B)
            pltpu.sync_copy(src_hbm.at[pl.ds(base, _B)], src_v)
            pltpu.sync_copy(dst_hbm.at[pl.ds(base, _B)], dst_v)
            pltpu.sync_copy(w_hbm.at[pl.ds(base, _B)], w_v)
            pltpu.async_copy(seq_hbm.at[src_v], rows_v, sem).wait()

            def group_body(g, carry2):
                wv = w_v[pl.ds(g * _LANES, _LANES)]
                for kk in range(_LANES):
                    w = wv[kk]
                    j = g * _LANES + kk
                    for k in range(_F // _LANES):
                        sl = pl.ds(k * _LANES, _LANES)
                        rows_v[j, sl] = rows_v[j, sl] * w
                return carry2

            lax.fori_loop(0, _B // _LANES, group_body, 0)
            pltpu.sync_copy(rows_v, acc.at[dst_v], add=True)
            return carry

        lax.fori_loop(0, nb, batch_body, 0)
        plsc.subcore_barrier()
        out_row0 = (c * 2 + sq) * n_nodes + s * rows_per_tile
        pltpu.sync_copy(acc.at[my_rows], out_hbm.at[pl.ds(out_row0, rows_per_tile)])


def _sc_aggregate(seq1, seq2, edge_index1, edge_weight1, edge_index2,
                  edge_weight2):
    n_real = seq1.shape[0]
    n_nodes = pl.cdiv(n_real, _TILES * 8) * _TILES * 8
    if n_nodes != n_real:
        seq1 = jnp.pad(seq1, ((0, n_nodes - n_real), (0, 0)))
        seq2 = jnp.pad(seq2, ((0, n_nodes - n_real), (0, 0)))
    e = edge_index1.shape[1]
    ept = pl.cdiv(e, _TILES * _B) * _B
    ep = ept * _TILES
    pad = ep - e

    def prep(ei, ew):
        src = jnp.pad(ei[0], (0, pad))
        dst = jnp.pad(ei[1], (0, pad))
        w = jnp.pad(ew, (0, pad))
        return src, dst, w

    s1, d1, w1 = prep(edge_index1, edge_weight1)
    s2, d2, w2 = prep(edge_index2, edge_weight2)
    src_all = jnp.concatenate([s1, s2])
    dst_all = jnp.concatenate([d1, d2])
    w_all = jnp.concatenate([w1, w2])
    zeros = jnp.zeros((n_nodes // _TILES, _F), jnp.float32)

    mesh = plsc.VectorSubcoreMesh(core_axis_name="c", subcore_axis_name="s")
    kern = functools.partial(
        pl.kernel,
        mesh=mesh,
        out_type=jax.ShapeDtypeStruct((4 * n_nodes, _F), jnp.float32),
        scratch_types=[
            pltpu.VMEM_SHARED((n_nodes, _F), jnp.float32),
            pltpu.VMEM((_B, _F), jnp.float32),
            pltpu.VMEM((_B,), jnp.int32),
            pltpu.VMEM((_B,), jnp.int32),
            pltpu.VMEM((_B,), jnp.float32),
            pltpu.SemaphoreType.DMA,
        ],
    )(functools.partial(_seg_body, n_nodes, ep, ept))
    agg = kern(seq1, seq2, src_all, dst_all, w_all, zeros)
    return agg.reshape(4, n_nodes, _F)[:, :n_real, :]


def _prelu(x, a):
    return jnp.where(x > 0, x, a * x)


def _bilinear_vec(W, b, x, cvec):
    return x @ (W @ cvec) + b[0]


def _dense_phase(agg, seq1, seq2, p):
    s_pl_p = jnp.tanh(seq1 @ p['disc_lin_W'] + p['disc_lin_b'])
    s_mi_p = jnp.tanh(seq2 @ p['disc_lin_W'] + p['disc_lin_b'])

    h1_list, h2_list, le, li, lj = [], [], [], [], []
    for i in range(2):
        W, b, a = p['gcn_W_%d' % i], p['gcn_b_%d' % i], p['gcn_prelu_%d' % i]
        h1 = _prelu(agg[2 * i] @ W + b, a)
        h2 = _prelu(agg[2 * i + 1] @ W + b, a)
        h1_list.append(h1)
        h2_list.append(h2)

    def disc(h_pl, h_mi):
        cvec = jnp.mean(h_pl, axis=0)
        logits_e = jnp.concatenate([
            _bilinear_vec(p['disc_Wk_e'], p['disc_bk_e'], h_pl, cvec),
            _bilinear_vec(p['disc_Wk_e'], p['disc_bk_e'], h_mi, cvec)])
        hW = h_pl @ p['disc_Wk_i']
        logits_i = jnp.concatenate([
            jnp.sum(hW * s_pl_p, axis=1) + p['disc_bk_i'][0],
            jnp.sum(hW * s_mi_p, axis=1) + p['disc_bk_i'][0]])
        logits_j = jnp.concatenate([
            _bilinear_vec(p['disc_Wk_j'], p['disc_bk_j'], s_pl_p, cvec),
            _bilinear_vec(p['disc_Wk_j'], p['disc_bk_j'], s_mi_p, cvec)])
        return logits_e, logits_i, logits_j

    for i in range(2):
        e_l, i_l, j_l = disc(h1_list[i], h2_list[i])
        le.append(e_l)
        li.append(i_l)
        lj.append(j_l)

    def combine(h_list):
        scores = []
        for i, h in enumerate(h_list):
            v = p['att_W_%d' % i] @ p['att_yW_%d' % i][:, 0]
            scores.append(h @ v + p['att_yb_%d' % i][0])
        score = jnp.stack(scores, axis=-1)
        score = jax.nn.softmax(jnp.tanh(score), axis=-1)
        return sum(score[:, i:i + 1] * h_list[i] for i in range(len(h_list)))

    h1 = combine(h1_list)
    h2 = combine(h2_list)
    e_f, i_f, j_f = disc(h1, h2)
    return (le[0], le[1], li[0], li[1], lj[0], lj[1], e_f, i_f, j_f)


def kernel(seq1, seq2, edge_index1, edge_weight1, edge_index2, edge_weight2,
           params, sparse):
    agg = _sc_aggregate(seq1, seq2, edge_index1, edge_weight1, edge_index2,
                        edge_weight2)
    return _dense_phase(agg, seq1, seq2, params)


# speedup vs baseline: 5.4712x; 1.3813x over previous
import functools

import jax
import jax.numpy as jnp
from jax import lax
from jax.experimental import pallas as pl
from jax.experimental.pallas import tpu as pltpu
from jax.experimental.pallas import tpu_sc as plsc

_F = 128
_TILES = 16
_B = 80
_LANES = 16
_R = 4
_BN = 2000


def _seg_body(n_nodes, nb, seq1_hbm, seq2_hbm, src_hbm, dst_hbm, w_hbm,
              zeros_hbm, out_hbm, acc, src_ring, dst_ring, w_ring,
              r0, r1, r2, r3, g0, g1, g2, g3, s0, s1, s2, s3,
              i0, i1, i2, i3):
    c = lax.axis_index("c")
    s = lax.axis_index("s")
    rows_buf = (r0, r1, r2, r3)
    gsem = (g0, g1, g2, g3)
    ssem = (s0, s1, s2, s3)
    isem = (i0, i1, i2, i3)
    rpt = n_nodes // _TILES
    my_rows = pl.ds(s * rpt, rpt)
    rbase = (c * _TILES + s) * nb

    def idx_copies(b, r):
        row = pl.ds(rbase + b, 1)
        slot = pl.ds(r, 1)
        return (
            pltpu.make_async_copy(src_hbm.at[row], src_ring.at[slot], isem[r]),
            pltpu.make_async_copy(dst_hbm.at[row], dst_ring.at[slot], isem[r]),
            pltpu.make_async_copy(w_hbm.at[row], w_ring.at[slot], isem[r]),
        )

    def start_idx(b, r):
        for cp in idx_copies(b, r):
            cp.start()

    def wait_idx(b, r):
        for cp in idx_copies(b, r):
            cp.wait()

    for sq, seq_hbm in enumerate((seq1_hbm, seq2_hbm)):
        pltpu.sync_copy(zeros_hbm.at[pl.ds(0, rpt)], acc.at[my_rows])
        plsc.subcore_barrier()

        def start_gather(r, seq_hbm=seq_hbm):
            pltpu.async_copy(seq_hbm.at[src_ring.at[r, 0]], rows_buf[r],
                             gsem[r])

        def wait_gather(r, seq_hbm=seq_hbm):
            pltpu.make_async_copy(seq_hbm.at[src_ring.at[r, 0]], rows_buf[r],
                                  gsem[r]).wait()

        def start_scatter(r):
            pltpu.async_copy(rows_buf[r], acc.at[dst_ring.at[r, 0]], ssem[r],
                             add=True)

        def wait_scatter(r):
            pltpu.make_async_copy(rows_buf[r], acc.at[dst_ring.at[r, 0]],
                                  ssem[r]).wait()

        def scale(r):
            rv = rows_buf[r]

            def group_body(g, carry):
                wv = w_ring[r, 0, pl.ds(g * _LANES, _LANES)]
                for kk in range(_LANES):
                    w = wv[kk]
                    j = g * _LANES + kk
                    for k in range(_F // _LANES):
                        sl = pl.ds(k * _LANES, _LANES)
                        rv[j, sl] = rv[j, sl] * w
                return carry

            lax.fori_loop(0, _B // _LANES, group_body, 0)

        for b0 in range(_R - 2):
            start_idx(b0, b0)
            wait_idx(b0, b0)
            start_gather(b0)

        def quad_body(i, carry):
            for r in range(_R):
                b = i * _R + r
                rn = (r + 2) % _R

                @pl.when(b >= 2)
                def _():
                    wait_scatter(rn)

                @pl.when(b + 2 < nb)
                def _():
                    start_idx(b + 2, rn)

                wait_gather(r)
                scale(r)
                start_scatter(r)

                @pl.when(b + 2 < nb)
                def _():
                    wait_idx(b + 2, rn)
                    start_gather(rn)
            return carry

        lax.fori_loop(0, nb // _R, quad_body, 0)
        for bt in range(nb - 2, nb):
            wait_scatter(bt % _R)
        plsc.subcore_barrier()
        out_row0 = (c * 2 + sq) * n_nodes + s * rpt
        pltpu.sync_copy(acc.at[my_rows], out_hbm.at[pl.ds(out_row0, rpt)])


def _sc_aggregate(seq1, seq2, edge_index1, edge_weight1, edge_index2,
                  edge_weight2):
    n_real = seq1.shape[0]
    n_nodes = pl.cdiv(n_real, _TILES * 8) * _TILES * 8
    if n_nodes != n_real:
        seq1 = jnp.pad(seq1, ((0, n_nodes - n_real), (0, 0)))
        seq2 = jnp.pad(seq2, ((0, n_nodes - n_real), (0, 0)))
    e = edge_index1.shape[1]
    nb = pl.cdiv(pl.cdiv(e, _TILES), _B)
    nb = pl.cdiv(nb, _R) * _R
    ept = nb * _B
    ep = ept * _TILES
    pad = ep - e

    def prep(ei, ew):
        src = jnp.pad(ei[0], (0, pad)).reshape(_TILES * nb, 1, _B)
        dst = jnp.pad(ei[1], (0, pad)).reshape(_TILES * nb, 1, _B)
        w = jnp.pad(ew, (0, pad)).reshape(_TILES * nb, 1, _B)
        return src, dst, w

    s1, d1, w1 = prep(edge_index1, edge_weight1)
    s2, d2, w2 = prep(edge_index2, edge_weight2)
    src_all = jnp.concatenate([s1, s2])
    dst_all = jnp.concatenate([d1, d2], axis=0)
    w_all = jnp.concatenate([w1, w2], axis=0)
    zeros = jnp.zeros((n_nodes // _TILES, _F), jnp.float32)

    mesh = plsc.VectorSubcoreMesh(core_axis_name="c", subcore_axis_name="s")
    kern = functools.partial(
        pl.kernel,
        mesh=mesh,
        out_type=jax.ShapeDtypeStruct((4 * n_nodes, _F), jnp.float32),
        scratch_types=[
            pltpu.VMEM_SHARED((n_nodes, _F), jnp.float32),
            pltpu.VMEM((_R, 1, _B), jnp.int32),
            pltpu.VMEM((_R, 1, _B), jnp.int32),
            pltpu.VMEM((_R, 1, _B), jnp.float32),
        ] + [pltpu.VMEM((_B, _F), jnp.float32)] * _R
          + [pltpu.SemaphoreType.DMA] * (3 * _R),
    )(functools.partial(_seg_body, n_nodes, nb))
    agg = kern(seq1, seq2, src_all, dst_all, w_all, zeros)
    return agg.reshape(4, n_nodes, _F)[:, :n_real, :]


def _prelu(x, a):
    return jnp.where(x > 0, x, a * x)


def _b1_body(agg0, agg1, agg2, agg3, seq1, seq2,
             W0, W1, linW, Wk_i, aW0, ayW0, aW1, ayW1, smalls,
             h10, h11, h20, h21, hc1, hc2, spp, smp, L1, csum):
    b0 = smalls[0:1, :]
    b1 = smalls[1:2, :]
    linb = smalls[2:3, :]
    a0 = smalls[3:4, 0:1]
    a1 = smalls[3:4, 1:2]
    yb0 = smalls[3:4, 2:3]
    yb1 = smalls[3:4, 3:4]
    bki = smalls[3:4, 4:5]

    f32 = jnp.float32
    dot = functools.partial(jnp.dot, preferred_element_type=f32)

    h1_0 = _prelu(dot(agg0[...], W0[...]) + b0, a0)
    h2_0 = _prelu(dot(agg1[...], W0[...]) + b0, a0)
    h1_1 = _prelu(dot(agg2[...], W1[...]) + b1, a1)
    h2_1 = _prelu(dot(agg3[...], W1[...]) + b1, a1)
    h10[...] = h1_0
    h20[...] = h2_0
    h11[...] = h1_1
    h21[...] = h2_1

    s_pl = jnp.tanh(dot(seq1[...], linW[...]) + linb)
    s_mi = jnp.tanh(dot(seq2[...], linW[...]) + linb)
    spp[...] = s_pl
    smp[...] = s_mi

    v0 = dot(aW0[...], ayW0[...])
    v1 = dot(aW1[...], ayW1[...])

    def combine(ha, hb):
        sc0 = jnp.tanh(dot(ha, v0) + yb0)
        sc1 = jnp.tanh(dot(hb, v1) + yb1)
        m = jnp.maximum(sc0, sc1)
        e0 = jnp.exp(sc0 - m)
        e1 = jnp.exp(sc1 - m)
        den = e0 + e1
        return (e0 / den) * ha + (e1 / den) * hb

    hc1_v = combine(h1_0, h1_1)
    hc2_v = combine(h2_0, h2_1)
    hc1[...] = hc1_v
    hc2[...] = hc2_v

    q0 = dot(h1_0, Wk_i[...])
    q1 = dot(h1_1, Wk_i[...])
    qf = dot(hc1_v, Wk_i[...])
    rsum = functools.partial(jnp.sum, axis=1, keepdims=True)
    vals = [
        rsum(q0 * s_pl) + bki,
        rsum(q0 * s_mi) + bki,
        rsum(q1 * s_pl) + bki,
        rsum(q1 * s_mi) + bki,
        rsum(qf * s_pl) + bki,
        rsum(qf * s_mi) + bki,
    ]
    bn = q0.shape[0]
    L1[...] = jnp.concatenate(
        vals + [jnp.zeros((bn, _F - len(vals)), f32)], axis=1)

    @pl.when(pl.program_id(0) == 0)
    def _():
        csum[...] = jnp.zeros_like(csum)

    csum[...] += jnp.concatenate([
        jnp.sum(h1_0, axis=0, keepdims=True),
        jnp.sum(h1_1, axis=0, keepdims=True),
        jnp.sum(hc1_v, axis=0, keepdims=True),
        jnp.zeros((5, _F), f32)], axis=0)


def _b2_body(n_nodes, h10, h11, h20, h21, hc1, hc2, spp, smp, csum,
             Wk_eT, Wk_jT, smalls, L2):
    bke = smalls[3:4, 5:6]
    bkj = smalls[3:4, 6:7]
    f32 = jnp.float32
    dot = functools.partial(jnp.dot, preferred_element_type=f32)
    inv_n = 1.0 / n_nodes
    c0 = csum[0:1, :] * inv_n
    c1 = csum[1:2, :] * inv_n
    cf = csum[2:3, :] * inv_n
    ue0 = dot(c0, Wk_eT[...])
    ue1 = dot(c1, Wk_eT[...])
    uef = dot(cf, Wk_eT[...])
    uj0 = dot(c0, Wk_jT[...])
    uj1 = dot(c1, Wk_jT[...])
    ujf = dot(cf, Wk_jT[...])
    rsum = functools.partial(jnp.sum, axis=1, keepdims=True)
    vals = [
        rsum(h10[...] * ue0) + bke,
        rsum(h20[...] * ue0) + bke,
        rsum(h11[...] * ue1) + bke,
        rsum(h21[...] * ue1) + bke,
        rsum(hc1[...] * uef) + bke,
        rsum(hc2[...] * uef) + bke,
        rsum(spp[...] * uj0) + bkj,
        rsum(smp[...] * uj0) + bkj,
        rsum(spp[...] * uj1) + bkj,
        rsum(smp[...] * uj1) + bkj,
        rsum(spp[...] * ujf) + bkj,
        rsum(smp[...] * ujf) + bkj,
    ]
    bn = vals[0].shape[0]
    L2[...] = jnp.concatenate(
        vals + [jnp.zeros((bn, _F - len(vals)), f32)], axis=1)


def _dense_phase(agg, seq1, seq2, p):
    n = seq1.shape[0]
    bn = _BN
    nb = n // bn
    f32 = jnp.float32

    smalls = jnp.zeros((8, _F), f32)
    smalls = smalls.at[0, :].set(p['gcn_b_0'])
    smalls = smalls.at[1, :].set(p['gcn_b_1'])
    smalls = smalls.at[2, :].set(p['disc_lin_b'])
    misc = jnp.stack([
        p['gcn_prelu_0'][0], p['gcn_prelu_1'][0],
        p['att_yb_0'][0], p['att_yb_1'][0],
        p['disc_bk_i'][0], p['disc_bk_e'][0], p['disc_bk_j'][0],
        jnp.float32(0.0)])
    smalls = smalls.at[3, :8].set(misc)

    blk = pl.BlockSpec((bn, _F), lambda i: (i, 0))
    wblk = pl.BlockSpec((_F, _F), lambda i: (0, 0))
    yblk = pl.BlockSpec((_F, 1), lambda i: (0, 0))
    sblk = pl.BlockSpec((8, _F), lambda i: (0, 0))
    nf = jax.ShapeDtypeStruct((n, _F), f32)

    outs = pl.pallas_call(
        _b1_body,
        grid=(nb,),
        in_specs=[blk] * 6 + [wblk] * 4 + [wblk, yblk, wblk, yblk] + [sblk],
        out_specs=[blk] * 9 + [sblk],
        out_shape=[nf] * 9 + [jax.ShapeDtypeStruct((8, _F), f32)],
    )(agg[0], agg[1], agg[2], agg[3], seq1, seq2,
      p['gcn_W_0'], p['gcn_W_1'], p['disc_lin_W'], p['disc_Wk_i'],
      p['att_W_0'], p['att_yW_0'], p['att_W_1'], p['att_yW_1'], smalls)
    h10, h11, h20, h21, hc1, hc2, spp, smp, L1, csum = outs

    L2 = pl.pallas_call(
        functools.partial(_b2_body, n),
        grid=(nb,),
        in_specs=[blk] * 8 + [sblk, wblk, wblk, sblk],
        out_specs=blk,
        out_shape=nf,
    )(h10, h11, h20, h21, hc1, hc2, spp, smp, csum,
      p['disc_Wk_e'].T, p['disc_Wk_j'].T, smalls)

    le0 = jnp.concatenate([L2[:, 0], L2[:, 1]])
    le1 = jnp.concatenate([L2[:, 2], L2[:, 3]])
    li0 = jnp.concatenate([L1[:, 0], L1[:, 1]])
    li1 = jnp.concatenate([L1[:, 2], L1[:, 3]])
    lj0 = jnp.concatenate([L2[:, 6], L2[:, 7]])
    lj1 = jnp.concatenate([L2[:, 8], L2[:, 9]])
    e_f = jnp.concatenate([L2[:, 4], L2[:, 5]])
    i_f = jnp.concatenate([L1[:, 4], L1[:, 5]])
    j_f = jnp.concatenate([L2[:, 10], L2[:, 11]])
    return (le0, le1, li0, li1, lj0, lj1, e_f, i_f, j_f)


def kernel(seq1, seq2, edge_index1, edge_weight1, edge_index2, edge_weight2,
           params, sparse):
    agg = _sc_aggregate(seq1, seq2, edge_index1, edge_weight1, edge_index2,
                        edge_weight2)
    return _dense_phase(agg, seq1, seq2, params)
